# baseline (speedup 1.0000x reference)
import functools

import jax
import jax.numpy as jnp
from jax import lax
from jax.experimental import pallas as pl
from jax.experimental.pallas import tpu as pltpu
from jax.experimental.pallas import tpu_sc as plsc

_NENT = 10000
_NUSR = 4096
_CH = 128
_NEDGE = 320000
_TOPK = 10
_LAM = 0.5

_NW = 32
_EPW = _NEDGE // _NW
_C = 80
_NCH = _EPW // _C
_SEC = 2000
_NSEC = _EPW // _SEC
_NCS = _SEC // _C

_ROWS = 256
_GRID = (_NENT + _ROWS - 1) // _ROWS
_KROWS = 128
_KGRID = (_NENT + _KROWS - 1) // _KROWS

_f32 = jnp.float32


def _sc_mesh():
    return plsc.VectorSubcoreMesh(
        core_axis_name="c", subcore_axis_name="s", num_cores=2, num_subcores=16)


def _gmat(e, w2p):
    def body(e_ref, w_ref, g_ref):
        x = e_ref[...]
        g_ref[...] = lax.dot_general(x * x, w_ref[...],
                                     (((1,), (1,)), ((), ())),
                                     preferred_element_type=_f32)
    return pl.pallas_call(
        body,
        grid=(_GRID,),
        in_specs=[pl.BlockSpec((_ROWS, _CH), lambda i: (i, 0)),
                  pl.BlockSpec((16, _CH), lambda i: (0, 0))],
        out_specs=pl.BlockSpec((_ROWS, 16), lambda i: (i, 0)),
        out_shape=jax.ShapeDtypeStruct((_NENT, 16), _f32),
    )(e, w2p)


def _posthop(y0, y1, res_in, w2p):
    def body(y0_ref, y1_ref, r_ref, w_ref, e_ref, ro_ref, g_ref):
        agg = y0_ref[...] + y1_ref[...]
        n = jnp.sqrt(jnp.sum(agg * agg, axis=1, keepdims=True))
        e = agg / jnp.maximum(n, 1e-12)
        e_ref[...] = e
        ro_ref[...] = r_ref[...] + e
        g_ref[...] = lax.dot_general(e * e, w_ref[...],
                                     (((1,), (1,)), ((), ())),
                                     preferred_element_type=_f32)
    return pl.pallas_call(
        body,
        grid=(_GRID,),
        in_specs=[pl.BlockSpec((_ROWS, _CH), lambda i: (i, 0)),
                  pl.BlockSpec((_ROWS, _CH), lambda i: (i, 0)),
                  pl.BlockSpec((_ROWS, _CH), lambda i: (i, 0)),
                  pl.BlockSpec((16, _CH), lambda i: (0, 0))],
        out_specs=[pl.BlockSpec((_ROWS, _CH), lambda i: (i, 0)),
                   pl.BlockSpec((_ROWS, _CH), lambda i: (i, 0)),
                   pl.BlockSpec((_ROWS, 16), lambda i: (i, 0))],
        out_shape=[jax.ShapeDtypeStruct((_NENT, _CH), _f32),
                   jax.ShapeDtypeStruct((_NENT, _CH), _f32),
                   jax.ShapeDtypeStruct((_NENT, 16), _f32)],
    )(y0, y1, res_in, w2p)


def _knn(ctx):
    def body(row_ref, full_ref, a_ref, rs_ref):
        a = row_ref[...]
        a = a * lax.rsqrt(jnp.sum(a * a, axis=1, keepdims=True))
        c = full_ref[...]
        c = c * lax.rsqrt(jnp.sum(c * c, axis=1, keepdims=True))
        sim = lax.dot_general(a, c, (((1,), (1,)), ((), ())),
                              preferred_element_type=_f32)
        work = sim
        sel = sim < _f32(-3.0)
        rowsum = jnp.zeros((sim.shape[0], 1), _f32)
        for _ in range(_TOPK):
            m = jnp.max(work, axis=1, keepdims=True)
            hit = work == m
            sel = jnp.logical_or(sel, hit)
            rowsum = rowsum + m
            work = jnp.where(hit, _f32(-3.0), work)
        a_ref[...] = jnp.where(sel, sim, _f32(0.0))
        rs_ref[...] = rowsum
    return pl.pallas_call(
        body,
        grid=(_KGRID,),
        in_specs=[pl.BlockSpec((_KROWS, _CH), lambda i: (i, 0)),
                  pl.BlockSpec((_NENT, _CH), lambda i: (0, 0))],
        out_specs=[pl.BlockSpec((_KROWS, _NENT), lambda i: (i, 0)),
                   pl.BlockSpec((_KROWS, 1), lambda i: (i, 0))],
        out_shape=[jax.ShapeDtypeStruct((_NENT, _NENT), _f32),
                   jax.ShapeDtypeStruct((_NENT, 1), _f32)],
    )(ctx, ctx)


def _combine(a1, a2, rs1r, rs1c, rs2r, rs2c):
    def body(a1_ref, a2_ref, r1r, r1c, r2r, r2c, o_ref):
        d1r = lax.rsqrt(r1r[...])
        d1c = lax.rsqrt(r1c[...])
        d2r = lax.rsqrt(r2r[...])
        d2c = lax.rsqrt(r2c[...])
        o_ref[...] = (_f32(1.0 - _LAM) * (d2r * a2_ref[...] * d2c)
                      + _f32(_LAM) * (d1r * a1_ref[...] * d1c))
    return pl.pallas_call(
        body,
        grid=(_KGRID,),
        in_specs=[pl.BlockSpec((_KROWS, _NENT), lambda i: (i, 0)),
                  pl.BlockSpec((_KROWS, _NENT), lambda i: (i, 0)),
                  pl.BlockSpec((_KROWS, 1), lambda i: (i, 0)),
                  pl.BlockSpec((1, _NENT), lambda i: (0, 0)),
                  pl.BlockSpec((_KROWS, 1), lambda i: (i, 0)),
                  pl.BlockSpec((1, _NENT), lambda i: (0, 0))],
        out_specs=pl.BlockSpec((_KROWS, _NENT), lambda i: (i, 0)),
        out_shape=jax.ShapeDtypeStruct((_NENT, _NENT), _f32),
    )(a1, a2, rs1r, rs1c, rs2r, rs2c)


def _vecadd(a, b):
    def body(a_ref, b_ref, o_ref):
        o_ref[...] = a_ref[...] + b_ref[...]
    return pl.pallas_call(
        body,
        out_shape=jax.ShapeDtypeStruct((8, 1250), _f32),
    )(a, b)


def _usermm(mat, ecat):
    def body(m_ref, e_ref, o_ref):
        o_ref[...] = lax.dot_general(m_ref[...], e_ref[...],
                                     (((1,), (0,)), ((), ())),
                                     preferred_element_type=_f32)
    return pl.pallas_call(
        body,
        grid=(16,),
        in_specs=[pl.BlockSpec((256, _NENT), lambda i: (i, 0)),
                  pl.BlockSpec((_NENT, 2 * _CH), lambda i: (0, 0))],
        out_specs=pl.BlockSpec((256, 2 * _CH), lambda i: (i, 0)),
        out_shape=jax.ShapeDtypeStruct((_NUSR, 2 * _CH), _f32),
    )(mat, ecat)


def _userupd(ua, u0, wp):
    def one(u, uagg, w):
        s = lax.dot_general(u, w, (((1,), (1,)), ((), ())),
                            preferred_element_type=_f32)
        col = lax.broadcasted_iota(jnp.int32, s.shape, 1)
        s = jnp.where(col < 10, s, _f32(-1e30))
        m = jnp.max(s, axis=1, keepdims=True)
        ex = jnp.exp(s - m)
        p = ex / jnp.sum(ex, axis=1, keepdims=True)
        sc = lax.dot_general(p, w, (((1,), (0,)), ((), ())),
                             preferred_element_type=_f32)
        v = uagg + sc * uagg
        n = jnp.sqrt(jnp.sum(v * v, axis=1, keepdims=True))
        return v / jnp.maximum(n, 1e-12)

    def body(ua_ref, u0_ref, w_ref, o_ref):
        w = w_ref[...]
        u0b = u0_ref[...]
        ua = ua_ref[...]
        u1 = one(u0b, ua[:, :_CH], w)
        u2 = one(u1, ua[:, _CH:], w)
        o_ref[...] = u0b + u1 + u2
    return pl.pallas_call(
        body,
        grid=(8,),
        in_specs=[pl.BlockSpec((512, 2 * _CH), lambda i: (i, 0)),
                  pl.BlockSpec((512, _CH), lambda i: (i, 0)),
                  pl.BlockSpec((16, _CH), lambda i: (0, 0))],
        out_specs=pl.BlockSpec((512, _CH), lambda i: (i, 0)),
        out_shape=jax.ShapeDtypeStruct((_NUSR, _CH), _f32),
    )(ua, u0, wp)


def _edge_a(idxa, idxb, h2d, gflat):
    def body(ia_hbm, ib_hbm, h2_hbm, g_hbm, att_hbm, s0_hbm, s1_hbm,
             ia_v, ib_v, ga_v, gb_v, att_v, h2_v, s_sh, sem):
        cid = lax.axis_index("c")
        sid = lax.axis_index("s")
        wid = cid * 16 + sid
        base = wid * _EPW
        zv = jnp.zeros((16,), _f32)

        def zb(i, c):
            att_v[pl.ds(i * 16, 16)] = zv
            return c
        lax.fori_loop(0, _EPW // 16, zb, 0)

        @pl.when(sid == 0)
        def _():
            pltpu.sync_copy(att_v, s_sh)
        plsc.subcore_barrier()

        pltpu.sync_copy(ia_hbm.at[pl.ds(base, _EPW)], ia_v)
        pltpu.sync_copy(ib_hbm.at[pl.ds(base, _EPW)], ib_v)
        pltpu.sync_copy(h2_hbm.at[wid], h2_v)
        pltpu.async_copy(g_hbm.at[ia_v], ga_v, sem).wait()
        pltpu.async_copy(g_hbm.at[ib_v], gb_v, sem).wait()

        def ab(i, c):
            sl = pl.ds(i * 16, 16)
            att_v[sl] = jnp.exp(ga_v[sl] * gb_v[sl])
            return c
        lax.fori_loop(0, _EPW // 16, ab, 0)

        pltpu.sync_copy(att_v, att_hbm.at[pl.ds(base, _EPW)])

        def sb(j, c):
            pltpu.sync_copy(att_v.at[pl.ds(j * _C, _C)],
                            s_sh.at[h2_v.at[j]], add=True)
            return c
        lax.fori_loop(0, _NCH, sb, 0)
        plsc.subcore_barrier()

        @pl.when((sid == 0) & (cid == 0))
        def _():
            pltpu.sync_copy(s_sh, s0_hbm)

        @pl.when((sid == 0) & (cid == 1))
        def _():
            pltpu.sync_copy(s_sh, s1_hbm)

    return pl.kernel(
        body,
        out_type=[jax.ShapeDtypeStruct((_NEDGE,), _f32),
                  jax.ShapeDtypeStruct((_NENT,), _f32),
                  jax.ShapeDtypeStruct((_NENT,), _f32)],
        mesh=_sc_mesh(),
        scratch_types=[pltpu.VMEM((_EPW,), jnp.int32),
                       pltpu.VMEM((_EPW,), jnp.int32),
                       pltpu.VMEM((_EPW,), _f32),
                       pltpu.VMEM((_EPW,), _f32),
                       pltpu.VMEM((_EPW,), _f32),
                       pltpu.VMEM((_NCH, _C), jnp.int32),
                       pltpu.VMEM_SHARED((_NENT,), _f32),
                       pltpu.SemaphoreType.DMA],
    )(idxa, idxb, h2d, gflat)


def _edge_c(h1d, t1d, r1d, h2d, att, s, emb, relflat):
    def body(h_hbm, t_hbm, r_hbm, h2_hbm, att_hbm, s_hbm,
             emb_hbm, rel_hbm, y0_hbm, y1_hbm,
             h_v, t_v, r_v, att_v, sg_v, h2_v, rel_v, emb_b0, emb_b1, y_b,
             y_sh, sem, sem_g0, sem_g1):
        cid = lax.axis_index("c")
        sid = lax.axis_index("s")
        wid = cid * 16 + sid
        base = wid * _EPW
        zv = jnp.zeros((16,), _f32)
        lane = jnp.arange(16, dtype=jnp.int32)

        def zb(j, c):
            def zg(g, c2):
                y_b[j, pl.ds(g * 16, 16)] = zv
                return c2
            return lax.fori_loop(0, 8, zg, c)
        lax.fori_loop(0, _C, zb, 0)

        @pl.when(sid == 0)
        def _():
            for k2 in range(8):
                pltpu.sync_copy(y_b, y_sh.at[pl.ds(k2 * _C, _C)])

        @pl.when(sid > 0)
        def _():
            st = 640 + (sid - 1) * 624
            for k2 in range(7):
                pltpu.sync_copy(y_b, y_sh.at[pl.ds(st + k2 * _C, _C)])
            pltpu.sync_copy(y_b.at[pl.ds(0, 64)],
                            y_sh.at[pl.ds(st + 560, 64)])
        plsc.subcore_barrier()

        pltpu.sync_copy(rel_hbm, rel_v)

        def compute(cb, emb_b):
            def grp(q, c2):
                w16 = att_v[pl.ds(cb + q * 16, 16)]
                rb16 = r_v[pl.ds(cb + q * 16, 16)] * 128
                for ee in range(16):
                    row = q * 16 + ee
                    wv = jnp.full((16,), w16[ee], _f32)
                    rbase = jnp.full((16,), rb16[ee], jnp.int32)
                    for g in range(8):
                        rel16 = plsc.load_gather(
                            rel_v, [rbase + (lane + g * 16)])
                        y_b[row, pl.ds(g * 16, 16)] = (
                            wv * emb_b[row, pl.ds(g * 16, 16)] * rel16)
                return c2
            lax.fori_loop(0, _C // 16, grp, 0)

        def section(k, c0):
            sb = base + k * _SEC
            pltpu.sync_copy(h_hbm.at[pl.ds(sb, _SEC)], h_v)
            pltpu.sync_copy(t_hbm.at[pl.ds(sb, _SEC)], t_v)
            pltpu.sync_copy(r_hbm.at[pl.ds(sb, _SEC)], r_v)
            pltpu.sync_copy(att_hbm.at[pl.ds(sb, _SEC)], att_v)
            pltpu.sync_copy(h2_hbm.at[wid, k], h2_v)
            pltpu.async_copy(s_hbm.at[h_v], sg_v, sem).wait()

            def wb(i, c):
                sl = pl.ds(i * 16, 16)
                att_v[sl] = att_v[sl] / sg_v[sl]
                return c
            lax.fori_loop(0, _SEC // 16, wb, 0)

            pltpu.async_copy(emb_hbm.at[t_v.at[pl.ds(0, _C)]],
                             emb_b0, sem_g0)

            def chunk2(j2, c):
                cb0 = (2 * j2) * _C
                cb1 = cb0 + _C
                pltpu.async_copy(emb_hbm.at[t_v.at[pl.ds(cb1, _C)]],
                                 emb_b1, sem_g1)
                pltpu.make_async_copy(
                    emb_hbm.at[t_v.at[pl.ds(cb0, _C)]], emb_b0, sem_g0).wait()
                compute(cb0, emb_b0)
                pltpu.sync_copy(y_b, y_sh.at[h2_v.at[2 * j2]], add=True)
                pltpu.async_copy(
                    emb_hbm.at[t_v.at[pl.ds(cb0 + 2 * _C, _C)]],
                    emb_b0, sem_g0)
                pltpu.make_async_copy(
                    emb_hbm.at[t_v.at[pl.ds(cb1, _C)]], emb_b1, sem_g1).wait()
                compute(cb1, emb_b1)
                pltpu.sync_copy(y_b, y_sh.at[h2_v.at[2 * j2 + 1]], add=True)
                return c
            lax.fori_loop(0, _NCS // 2, chunk2, 0)
            tb = (_NCS - 1) * _C
            pltpu.make_async_copy(
                emb_hbm.at[t_v.at[pl.ds(tb, _C)]], emb_b0, sem_g0).wait()
            compute(tb, emb_b0)
            pltpu.sync_copy(y_b, y_sh.at[h2_v.at[_NCS - 1]], add=True)
            return c0
        lax.fori_loop(0, _EPW // _SEC, section, 0)
        plsc.subcore_barrier()

        @pl.when((cid == 0) & (sid == 0))
        def _():
            pltpu.sync_copy(y_sh.at[pl.ds(0, 640)], y0_hbm.at[pl.ds(0, 640)])

        @pl.when((cid == 0) & (sid > 0))
        def _():
            st = 640 + (sid - 1) * 624
            pltpu.sync_copy(y_sh.at[pl.ds(st, 624)],
                            y0_hbm.at[pl.ds(st, 624)])

        @pl.when((cid == 1) & (sid == 0))
        def _():
            pltpu.sync_copy(y_sh.at[pl.ds(0, 640)], y1_hbm.at[pl.ds(0, 640)])

        @pl.when((cid == 1) & (sid > 0))
        def _():
            st = 640 + (sid - 1) * 624
            pltpu.sync_copy(y_sh.at[pl.ds(st, 624)],
                            y1_hbm.at[pl.ds(st, 624)])

    return pl.kernel(
        body,
        out_type=[jax.ShapeDtypeStruct((_NENT, _CH), _f32),
                  jax.ShapeDtypeStruct((_NENT, _CH), _f32)],
        mesh=_sc_mesh(),
        scratch_types=[pltpu.VMEM((_SEC,), jnp.int32),
                       pltpu.VMEM((_SEC,), jnp.int32),
                       pltpu.VMEM((_SEC,), jnp.int32),
                       pltpu.VMEM((_SEC,), _f32),
                       pltpu.VMEM((_SEC,), _f32),
                       pltpu.VMEM((_NCS, _C), jnp.int32),
                       pltpu.VMEM((16 * _CH,), _f32),
                       pltpu.VMEM((_C, _CH), _f32),
                       pltpu.VMEM((_C, _CH), _f32),
                       pltpu.VMEM((_C, _CH), _f32),
                       pltpu.VMEM_SHARED((_NENT, _CH), _f32),
                       pltpu.SemaphoreType.DMA,
                       pltpu.SemaphoreType.DMA,
                       pltpu.SemaphoreType.DMA],
        compiler_params=pltpu.CompilerParams(needs_layout_passes=False),
    )(h1d, t1d, r1d, h2d, att, s, emb, relflat)


def _hop(e, h1d, t1d, r1d, h2a, h2c, idxa, idxb, gflat, relflat):
    att, s0, s1 = _edge_a(idxa, idxb, h2a, gflat)
    s = _vecadd(s0.reshape(8, 1250), s1.reshape(8, 1250)).reshape(-1)
    y0, y1 = _edge_c(h1d, t1d, r1d, h2c, att, s, e, relflat)
    return y0, y1


def kernel(user_emb, entity_emb, edge_index, edge_type, interact_mat, weight):
    h = edge_index[0].astype(jnp.int32)
    t = edge_index[1].astype(jnp.int32)
    et = edge_type.astype(jnp.int32)
    r = jnp.where(et >= 1, et - 1, et + 9)
    idxa = h * 16 + r
    idxb = t * 16 + r
    h2a = h.reshape(_NW, _NCH, _C)
    h2c = h.reshape(_NW, _NSEC, _NCS, _C)

    wp = jnp.zeros((16, _CH), _f32).at[:10].set(weight)
    w2p = wp * wp
    relflat = wp.reshape(-1)

    e0 = entity_emb
    g1 = _gmat(e0, w2p)
    y0, y1 = _hop(e0, h1d=h, t1d=t, r1d=r, h2a=h2a, h2c=h2c,
                  idxa=idxa, idxb=idxb, gflat=g1.reshape(-1), relflat=relflat)
    e1, res1, g2 = _posthop(y0, y1, e0, w2p)
    y0b, y1b = _hop(e1, h1d=h, t1d=t, r1d=r, h2a=h2a, h2c=h2c,
                    idxa=idxa, idxb=idxb, gflat=g2.reshape(-1), relflat=relflat)
    _, entity_res, _ = _posthop(y0b, y1b, res1, w2p)

    ua = _usermm(interact_mat, jnp.concatenate([e0, e1], axis=1))
    user_res = _userupd(ua, user_emb, wp)

    a1, rs1 = _knn(e0)
    a2, rs2 = _knn(entity_res)
    item_adj = _combine(a1, a2, rs1, rs1.reshape(1, _NENT),
                        rs2, rs2.reshape(1, _NENT))
    return entity_res, user_res, item_adj


# speedup vs baseline: 1.4179x; 1.4179x over previous
import functools

import jax
import jax.numpy as jnp
from jax import lax
from jax.experimental import pallas as pl
from jax.experimental.pallas import tpu as pltpu
from jax.experimental.pallas import tpu_sc as plsc

_NENT = 10000
_NUSR = 4096
_CH = 128
_NEDGE = 320000
_TOPK = 10
_LAM = 0.5

_NW = 32
_EPW = _NEDGE // _NW
_C = 80
_NCH = _EPW // _C
_SEC = 2000
_NSEC = _EPW // _SEC
_NCS = _SEC // _C

_ROWS = 256
_GRID = (_NENT + _ROWS - 1) // _ROWS
_KROWS = 128
_KGRID = (_NENT + _KROWS - 1) // _KROWS

_f32 = jnp.float32


def _sc_mesh():
    return plsc.VectorSubcoreMesh(
        core_axis_name="c", subcore_axis_name="s", num_cores=2, num_subcores=16)


def _gmat(e, w2p):
    def body(e_ref, w_ref, g_ref):
        x = e_ref[...]
        g_ref[...] = lax.dot_general(x * x, w_ref[...],
                                     (((1,), (1,)), ((), ())),
                                     preferred_element_type=_f32)
    return pl.pallas_call(
        body,
        grid=(_GRID,),
        in_specs=[pl.BlockSpec((_ROWS, _CH), lambda i: (i, 0)),
                  pl.BlockSpec((16, _CH), lambda i: (0, 0))],
        out_specs=pl.BlockSpec((_ROWS, 16), lambda i: (i, 0)),
        out_shape=jax.ShapeDtypeStruct((_NENT, 16), _f32),
    )(e, w2p)


def _posthop(y0, y1, res_in, w2p):
    def body(y0_ref, y1_ref, r_ref, w_ref, e_ref, ro_ref, g_ref):
        agg = y0_ref[...] + y1_ref[...]
        n = jnp.sqrt(jnp.sum(agg * agg, axis=1, keepdims=True))
        e = agg / jnp.maximum(n, 1e-12)
        e_ref[...] = e
        ro_ref[...] = r_ref[...] + e
        g_ref[...] = lax.dot_general(e * e, w_ref[...],
                                     (((1,), (1,)), ((), ())),
                                     preferred_element_type=_f32)
    return pl.pallas_call(
        body,
        grid=(_GRID,),
        in_specs=[pl.BlockSpec((_ROWS, _CH), lambda i: (i, 0)),
                  pl.BlockSpec((_ROWS, _CH), lambda i: (i, 0)),
                  pl.BlockSpec((_ROWS, _CH), lambda i: (i, 0)),
                  pl.BlockSpec((16, _CH), lambda i: (0, 0))],
        out_specs=[pl.BlockSpec((_ROWS, _CH), lambda i: (i, 0)),
                   pl.BlockSpec((_ROWS, _CH), lambda i: (i, 0)),
                   pl.BlockSpec((_ROWS, 16), lambda i: (i, 0))],
        out_shape=[jax.ShapeDtypeStruct((_NENT, _CH), _f32),
                   jax.ShapeDtypeStruct((_NENT, _CH), _f32),
                   jax.ShapeDtypeStruct((_NENT, 16), _f32)],
    )(y0, y1, res_in, w2p)


def _knn(ctx):
    def body(row_ref, full_ref, a_ref, rs_ref):
        a = row_ref[...]
        a = a * lax.rsqrt(jnp.sum(a * a, axis=1, keepdims=True))
        c = full_ref[...]
        c = c * lax.rsqrt(jnp.sum(c * c, axis=1, keepdims=True))
        sim = lax.dot_general(a, c, (((1,), (1,)), ((), ())),
                              preferred_element_type=_f32)
        work = sim
        rowsum = jnp.zeros((sim.shape[0], 1), _f32)
        for _ in range(_TOPK):
            m = jnp.max(work, axis=1, keepdims=True)
            rowsum = rowsum + m
            work = jnp.where(work == m, _f32(-3.0), work)
        a_ref[...] = jnp.where(work == _f32(-3.0), sim, _f32(0.0))
        rs_ref[...] = rowsum
    return pl.pallas_call(
        body,
        grid=(_KGRID,),
        in_specs=[pl.BlockSpec((_KROWS, _CH), lambda i: (i, 0)),
                  pl.BlockSpec((_NENT, _CH), lambda i: (0, 0))],
        out_specs=[pl.BlockSpec((_KROWS, _NENT), lambda i: (i, 0)),
                   pl.BlockSpec((_KROWS, 1), lambda i: (i, 0))],
        out_shape=[jax.ShapeDtypeStruct((_NENT, _NENT), _f32),
                   jax.ShapeDtypeStruct((_NENT, 1), _f32)],
    )(ctx, ctx)


def _combine(a1, a2, rs1r, rs1c, rs2r, rs2c):
    def body(a1_ref, a2_ref, r1r, r1c, r2r, r2c, o_ref):
        d1r = lax.rsqrt(r1r[...])
        d1c = lax.rsqrt(r1c[...])
        d2r = lax.rsqrt(r2r[...])
        d2c = lax.rsqrt(r2c[...])
        o_ref[...] = (_f32(1.0 - _LAM) * (d2r * a2_ref[...] * d2c)
                      + _f32(_LAM) * (d1r * a1_ref[...] * d1c))
    return pl.pallas_call(
        body,
        grid=(_KGRID,),
        in_specs=[pl.BlockSpec((_KROWS, _NENT), lambda i: (i, 0)),
                  pl.BlockSpec((_KROWS, _NENT), lambda i: (i, 0)),
                  pl.BlockSpec((_KROWS, 1), lambda i: (i, 0)),
                  pl.BlockSpec((1, _NENT), lambda i: (0, 0)),
                  pl.BlockSpec((_KROWS, 1), lambda i: (i, 0)),
                  pl.BlockSpec((1, _NENT), lambda i: (0, 0))],
        out_specs=pl.BlockSpec((_KROWS, _NENT), lambda i: (i, 0)),
        out_shape=jax.ShapeDtypeStruct((_NENT, _NENT), _f32),
    )(a1, a2, rs1r, rs1c, rs2r, rs2c)


def _vecadd(a, b):
    def body(a_ref, b_ref, o_ref):
        o_ref[...] = a_ref[...] + b_ref[...]
    return pl.pallas_call(
        body,
        out_shape=jax.ShapeDtypeStruct((8, 1250), _f32),
    )(a, b)


def _usermm(mat, ecat):
    def body(m_ref, e_ref, o_ref):
        o_ref[...] = lax.dot_general(m_ref[...], e_ref[...],
                                     (((1,), (0,)), ((), ())),
                                     preferred_element_type=_f32)
    return pl.pallas_call(
        body,
        grid=(16,),
        in_specs=[pl.BlockSpec((256, _NENT), lambda i: (i, 0)),
                  pl.BlockSpec((_NENT, 2 * _CH), lambda i: (0, 0))],
        out_specs=pl.BlockSpec((256, 2 * _CH), lambda i: (i, 0)),
        out_shape=jax.ShapeDtypeStruct((_NUSR, 2 * _CH), _f32),
    )(mat, ecat)


def _userupd(ua, u0, wp):
    def one(u, uagg, w):
        s = lax.dot_general(u, w, (((1,), (1,)), ((), ())),
                            preferred_element_type=_f32)
        col = lax.broadcasted_iota(jnp.int32, s.shape, 1)
        s = jnp.where(col < 10, s, _f32(-1e30))
        m = jnp.max(s, axis=1, keepdims=True)
        ex = jnp.exp(s - m)
        p = ex / jnp.sum(ex, axis=1, keepdims=True)
        sc = lax.dot_general(p, w, (((1,), (0,)), ((), ())),
                             preferred_element_type=_f32)
        v = uagg + sc * uagg
        n = jnp.sqrt(jnp.sum(v * v, axis=1, keepdims=True))
        return v / jnp.maximum(n, 1e-12)

    def body(ua_ref, u0_ref, w_ref, o_ref):
        w = w_ref[...]
        u0b = u0_ref[...]
        ua = ua_ref[...]
        u1 = one(u0b, ua[:, :_CH], w)
        u2 = one(u1, ua[:, _CH:], w)
        o_ref[...] = u0b + u1 + u2
    return pl.pallas_call(
        body,
        grid=(8,),
        in_specs=[pl.BlockSpec((512, 2 * _CH), lambda i: (i, 0)),
                  pl.BlockSpec((512, _CH), lambda i: (i, 0)),
                  pl.BlockSpec((16, _CH), lambda i: (0, 0))],
        out_specs=pl.BlockSpec((512, _CH), lambda i: (i, 0)),
        out_shape=jax.ShapeDtypeStruct((_NUSR, _CH), _f32),
    )(ua, u0, wp)


def _edge_a(idxa, idxb, h2d, gflat):
    def body(ia_hbm, ib_hbm, h2_hbm, g_hbm, att_hbm, s0_hbm, s1_hbm,
             ia_v, ib_v, ga_v, gb_v, att_v, h2_v, s_sh, sem):
        cid = lax.axis_index("c")
        sid = lax.axis_index("s")
        wid = cid * 16 + sid
        base = wid * _EPW
        zv = jnp.zeros((16,), _f32)

        def zb(i, c):
            att_v[pl.ds(i * 16, 16)] = zv
            return c
        lax.fori_loop(0, _EPW // 16, zb, 0)

        @pl.when(sid == 0)
        def _():
            pltpu.sync_copy(att_v, s_sh)
        plsc.subcore_barrier()

        pltpu.sync_copy(ia_hbm.at[pl.ds(base, _EPW)], ia_v)
        pltpu.sync_copy(ib_hbm.at[pl.ds(base, _EPW)], ib_v)
        pltpu.sync_copy(h2_hbm.at[wid], h2_v)
        pltpu.async_copy(g_hbm.at[ia_v], ga_v, sem).wait()
        pltpu.async_copy(g_hbm.at[ib_v], gb_v, sem).wait()

        def ab(i, c):
            sl = pl.ds(i * 16, 16)
            att_v[sl] = jnp.exp(ga_v[sl] * gb_v[sl])
            return c
        lax.fori_loop(0, _EPW // 16, ab, 0)

        pltpu.sync_copy(att_v, att_hbm.at[pl.ds(base, _EPW)])

        def sb(j, c):
            pltpu.sync_copy(att_v.at[pl.ds(j * _C, _C)],
                            s_sh.at[h2_v.at[j]], add=True)
            return c
        lax.fori_loop(0, _NCH, sb, 0)
        plsc.subcore_barrier()

        @pl.when((sid == 0) & (cid == 0))
        def _():
            pltpu.sync_copy(s_sh, s0_hbm)

        @pl.when((sid == 0) & (cid == 1))
        def _():
            pltpu.sync_copy(s_sh, s1_hbm)

    return pl.kernel(
        body,
        out_type=[jax.ShapeDtypeStruct((_NEDGE,), _f32),
                  jax.ShapeDtypeStruct((_NENT,), _f32),
                  jax.ShapeDtypeStruct((_NENT,), _f32)],
        mesh=_sc_mesh(),
        scratch_types=[pltpu.VMEM((_EPW,), jnp.int32),
                       pltpu.VMEM((_EPW,), jnp.int32),
                       pltpu.VMEM((_EPW,), _f32),
                       pltpu.VMEM((_EPW,), _f32),
                       pltpu.VMEM((_EPW,), _f32),
                       pltpu.VMEM((_NCH, _C), jnp.int32),
                       pltpu.VMEM_SHARED((_NENT,), _f32),
                       pltpu.SemaphoreType.DMA],
    )(idxa, idxb, h2d, gflat)


def _edge_c(h1d, t1d, r1d, h2d, att, s, emb, relp):
    def body(h_hbm, t_hbm, r_hbm, h2_hbm, att_hbm, s_hbm,
             emb_hbm, rel_hbm, y0_hbm, y1_hbm,
             h_v, t_v, r_v, att_v, sg_v, h2_v, rel_v, emb_b0, emb_b1, y_b,
             y_sh, sem, sem_g0, sem_g1):
        cid = lax.axis_index("c")
        sid = lax.axis_index("s")
        wid = cid * 16 + sid
        base = wid * _EPW
        zv = jnp.zeros((16,), _f32)

        def zb(j, c):
            def zg(g, c2):
                y_b[j, pl.ds(g * 16, 16)] = zv
                return c2
            return lax.fori_loop(0, 8, zg, c)
        lax.fori_loop(0, _C, zb, 0)

        @pl.when(sid == 0)
        def _():
            for k2 in range(8):
                pltpu.sync_copy(y_b, y_sh.at[pl.ds(k2 * _C, _C)])

        @pl.when(sid > 0)
        def _():
            st = 640 + (sid - 1) * 624
            for k2 in range(7):
                pltpu.sync_copy(y_b, y_sh.at[pl.ds(st + k2 * _C, _C)])
            pltpu.sync_copy(y_b.at[pl.ds(0, 64)],
                            y_sh.at[pl.ds(st + 560, 64)])
        plsc.subcore_barrier()

        pltpu.sync_copy(rel_hbm, rel_v)

        def compute(cb, emb_b):
            def grp(q, c2):
                w16 = att_v[pl.ds(cb + q * 16, 16)]
                r16 = r_v[pl.ds(cb + q * 16, 16)]
                for ee in range(16):
                    row = q * 16 + ee
                    wv = jnp.full((16,), w16[ee], _f32)
                    rr = r16[ee]
                    for g in range(8):
                        rel16 = rel_v[rr, pl.ds(g * 16, 16)]
                        y_b[row, pl.ds(g * 16, 16)] = (
                            wv * emb_b[row, pl.ds(g * 16, 16)] * rel16)
                return c2
            lax.fori_loop(0, _C // 16, grp, 0)

        def section(k, c0):
            sb = base + k * _SEC
            pltpu.sync_copy(h_hbm.at[pl.ds(sb, _SEC)], h_v)
            pltpu.sync_copy(t_hbm.at[pl.ds(sb, _SEC)], t_v)
            pltpu.sync_copy(r_hbm.at[pl.ds(sb, _SEC)], r_v)
            pltpu.sync_copy(att_hbm.at[pl.ds(sb, _SEC)], att_v)
            pltpu.sync_copy(h2_hbm.at[wid, k], h2_v)
            pltpu.async_copy(s_hbm.at[h_v], sg_v, sem).wait()

            def wb(i, c):
                sl = pl.ds(i * 16, 16)
                att_v[sl] = att_v[sl] / sg_v[sl]
                return c
            lax.fori_loop(0, _SEC // 16, wb, 0)

            pltpu.async_copy(emb_hbm.at[t_v.at[pl.ds(0, _C)]],
                             emb_b0, sem_g0)

            def chunk2(j2, c):
                cb0 = (2 * j2) * _C
                cb1 = cb0 + _C
                pltpu.async_copy(emb_hbm.at[t_v.at[pl.ds(cb1, _C)]],
                                 emb_b1, sem_g1)
                pltpu.make_async_copy(
                    emb_hbm.at[t_v.at[pl.ds(cb0, _C)]], emb_b0, sem_g0).wait()
                compute(cb0, emb_b0)
                pltpu.sync_copy(y_b, y_sh.at[h2_v.at[2 * j2]], add=True)
                pltpu.async_copy(
                    emb_hbm.at[t_v.at[pl.ds(cb0 + 2 * _C, _C)]],
                    emb_b0, sem_g0)
                pltpu.make_async_copy(
                    emb_hbm.at[t_v.at[pl.ds(cb1, _C)]], emb_b1, sem_g1).wait()
                compute(cb1, emb_b1)
                pltpu.sync_copy(y_b, y_sh.at[h2_v.at[2 * j2 + 1]], add=True)
                return c
            lax.fori_loop(0, _NCS // 2, chunk2, 0)
            tb = (_NCS - 1) * _C
            pltpu.make_async_copy(
                emb_hbm.at[t_v.at[pl.ds(tb, _C)]], emb_b0, sem_g0).wait()
            compute(tb, emb_b0)
            pltpu.sync_copy(y_b, y_sh.at[h2_v.at[_NCS - 1]], add=True)
            return c0
        lax.fori_loop(0, _EPW // _SEC, section, 0)
        plsc.subcore_barrier()

        @pl.when((cid == 0) & (sid == 0))
        def _():
            pltpu.sync_copy(y_sh.at[pl.ds(0, 640)], y0_hbm.at[pl.ds(0, 640)])

        @pl.when((cid == 0) & (sid > 0))
        def _():
            st = 640 + (sid - 1) * 624
            pltpu.sync_copy(y_sh.at[pl.ds(st, 624)],
                            y0_hbm.at[pl.ds(st, 624)])

        @pl.when((cid == 1) & (sid == 0))
        def _():
            pltpu.sync_copy(y_sh.at[pl.ds(0, 640)], y1_hbm.at[pl.ds(0, 640)])

        @pl.when((cid == 1) & (sid > 0))
        def _():
            st = 640 + (sid - 1) * 624
            pltpu.sync_copy(y_sh.at[pl.ds(st, 624)],
                            y1_hbm.at[pl.ds(st, 624)])

    return pl.kernel(
        body,
        out_type=[jax.ShapeDtypeStruct((_NENT, _CH), _f32),
                  jax.ShapeDtypeStruct((_NENT, _CH), _f32)],
        mesh=_sc_mesh(),
        scratch_types=[pltpu.VMEM((_SEC,), jnp.int32),
                       pltpu.VMEM((_SEC,), jnp.int32),
                       pltpu.VMEM((_SEC,), jnp.int32),
                       pltpu.VMEM((_SEC,), _f32),
                       pltpu.VMEM((_SEC,), _f32),
                       pltpu.VMEM((_NCS, _C), jnp.int32),
                       pltpu.VMEM((16, _CH), _f32),
                       pltpu.VMEM((_C, _CH), _f32),
                       pltpu.VMEM((_C, _CH), _f32),
                       pltpu.VMEM((_C, _CH), _f32),
                       pltpu.VMEM_SHARED((_NENT, _CH), _f32),
                       pltpu.SemaphoreType.DMA,
                       pltpu.SemaphoreType.DMA,
                       pltpu.SemaphoreType.DMA],
        compiler_params=pltpu.CompilerParams(needs_layout_passes=False),
    )(h1d, t1d, r1d, h2d, att, s, emb, relp)


def _hop(e, h1d, t1d, r1d, h2a, h2c, idxa, idxb, gflat, relp):
    att, s0, s1 = _edge_a(idxa, idxb, h2a, gflat)
    s = _vecadd(s0.reshape(8, 1250), s1.reshape(8, 1250)).reshape(-1)
    y0, y1 = _edge_c(h1d, t1d, r1d, h2c, att, s, e, relp)
    return y0, y1


def kernel(user_emb, entity_emb, edge_index, edge_type, interact_mat, weight):
    h = edge_index[0].astype(jnp.int32)
    t = edge_index[1].astype(jnp.int32)
    et = edge_type.astype(jnp.int32)
    r = jnp.where(et >= 1, et - 1, et + 9)
    idxa = h * 16 + r
    idxb = t * 16 + r
    h2a = h.reshape(_NW, _NCH, _C)
    h2c = h.reshape(_NW, _NSEC, _NCS, _C)

    wp = jnp.zeros((16, _CH), _f32).at[:10].set(weight)
    w2p = wp * wp

    e0 = entity_emb
    g1 = _gmat(e0, w2p)
    y0, y1 = _hop(e0, h1d=h, t1d=t, r1d=r, h2a=h2a, h2c=h2c,
                  idxa=idxa, idxb=idxb, gflat=g1.reshape(-1), relp=wp)
    e1, res1, g2 = _posthop(y0, y1, e0, w2p)
    y0b, y1b = _hop(e1, h1d=h, t1d=t, r1d=r, h2a=h2a, h2c=h2c,
                    idxa=idxa, idxb=idxb, gflat=g2.reshape(-1), relp=wp)
    _, entity_res, _ = _posthop(y0b, y1b, res1, w2p)

    ua = _usermm(interact_mat, jnp.concatenate([e0, e1], axis=1))
    user_res = _userupd(ua, user_emb, wp)

    a1, rs1 = _knn(e0)
    a2, rs2 = _knn(entity_res)
    item_adj = _combine(a1, a2, rs1, rs1.reshape(1, _NENT),
                        rs2, rs2.reshape(1, _NENT))
    return entity_res, user_res, item_adj
